# baseline (speedup 1.0000x reference)
import functools

import jax
import jax.numpy as jnp
from jax import lax
from jax.experimental import pallas as pl
from jax.experimental.pallas import tpu as pltpu
from jax.experimental.pallas import tpu_sc as plsc

N = 10000
E = 320000
D = 128
H = 256

NC = 2
NS = 16
NW = NC * NS
C = 128
SBC = 20
EPT = -(-(E // NW) // (C * SBC)) * (C * SBC)
NCHUNK = EPT // C
NSB = NCHUNK // SBC
EPAD = EPT * NW
ZROWS = 632
NSH = ZROWS * NS
NPAD_ROWS = NSH - N
ORECS = 624
OREM = N - ORECS * NS
BR = 1000
NBLK = N // BR


def _sc_body(x_hbm, srcp_hbm, dstp_hbm, zeros_hbm, out_hbm,
             sidx_a, didx_a, sidx_b, didx_b, rows_v, rows1_v, agg_sh,
             gsem, gsem1, isem_a, isem_b, zsem, ssem0, ssem1):
    cid = lax.axis_index("c")
    sid = lax.axis_index("s")
    wid = cid * NS + sid

    pltpu.async_copy(srcp_hbm.at[wid, 0], sidx_a, isem_a)
    pltpu.async_copy(dstp_hbm.at[wid, 0], didx_a, isem_a)
    pltpu.async_copy(srcp_hbm.at[wid, 1], sidx_b, isem_b)
    pltpu.async_copy(dstp_hbm.at[wid, 1], didx_b, isem_b)

    base = sid * ZROWS
    zcopies = []
    pltpu.sync_copy(zeros_hbm.at[sid], rows1_v)
    for k in range(ZROWS // C):
        zcopies.append((rows1_v, agg_sh.at[pl.ds(base + k * C, C)]))
    rem = ZROWS % C
    if rem:
        zcopies.append((rows1_v.at[pl.ds(0, rem)],
                        agg_sh.at[pl.ds(base + (ZROWS // C) * C, rem)]))
    for s, d in zcopies:
        pltpu.async_copy(s, d, zsem)
    for s, d in zcopies:
        pltpu.make_async_copy(s, d, zsem).wait()
    plsc.subcore_barrier()

    def swait(sem, didx_v):
        pltpu.make_async_copy(rows_v, agg_sh.at[didx_v.at[0]], sem).wait()

    def process(sidx_v, didx_v, isem, sb, restage, nsb):
        pltpu.make_async_copy(srcp_hbm.at[wid, sb], sidx_v, isem).wait()
        pltpu.make_async_copy(dstp_hbm.at[wid, sb], didx_v, isem).wait()
        if sb > 0:
            swait(ssem0, didx_v)
        pltpu.async_copy(x_hbm.at[sidx_v.at[0]], rows_v, gsem)
        if sb > 0:
            swait(ssem1, didx_v)
        pltpu.async_copy(x_hbm.at[sidx_v.at[1]], rows1_v, gsem1)
        if restage is not None:
            nsidx, ndidx, nisem = restage
            pltpu.async_copy(srcp_hbm.at[wid, nsb], nsidx, nisem)
            pltpu.async_copy(dstp_hbm.at[wid, nsb], ndidx, nisem)

        def chunk_pair(i, _):
            j = 2 * i
            pltpu.make_async_copy(x_hbm.at[sidx_v.at[j]], rows_v, gsem).wait()
            pltpu.async_copy(rows_v, agg_sh.at[didx_v.at[j]], ssem0, add=True)
            pltpu.make_async_copy(
                x_hbm.at[sidx_v.at[j + 1]], rows1_v, gsem1).wait()
            pltpu.async_copy(rows1_v, agg_sh.at[didx_v.at[j + 1]], ssem1,
                             add=True)

            @pl.when(j + 2 < SBC)
            def _g0():
                swait(ssem0, didx_v)
                pltpu.async_copy(x_hbm.at[sidx_v.at[j + 2]], rows_v, gsem)

            @pl.when(j + 3 < SBC)
            def _g1():
                swait(ssem1, didx_v)
                pltpu.async_copy(x_hbm.at[sidx_v.at[j + 3]], rows1_v, gsem1)
            return _

        lax.fori_loop(0, SBC // 2, chunk_pair, None)

    banks = [(sidx_a, didx_a, isem_a), (sidx_b, didx_b, isem_b)]
    for sb in range(NSB):
        sidx_v, didx_v, isem = banks[sb % 2]
        if 1 <= sb <= NSB - 2:
            restage, nsb = banks[(sb + 1) % 2], sb + 1
        else:
            restage, nsb = None, None
        process(sidx_v, didx_v, isem, sb, restage, nsb)
    swait(ssem0, didx_b if (NSB - 1) % 2 else didx_a)
    swait(ssem1, didx_b if (NSB - 1) % 2 else didx_a)
    plsc.subcore_barrier()

    pltpu.sync_copy(agg_sh.at[pl.ds(sid * ORECS, ORECS)],
                    out_hbm.at[cid, pl.ds(sid * ORECS, ORECS)])

    @pl.when(sid == NS - 1)
    def _tail():
        pltpu.sync_copy(agg_sh.at[pl.ds(ORECS * NS, OREM)],
                        out_hbm.at[cid, pl.ds(ORECS * NS, OREM)])


_sc_aggregate = functools.partial(
    pl.kernel,
    out_type=jax.ShapeDtypeStruct((NC, N, D), jnp.float32),
    mesh=plsc.VectorSubcoreMesh(core_axis_name="c", subcore_axis_name="s"),
    scratch_types=[
        pltpu.VMEM((SBC, C), jnp.int32),
        pltpu.VMEM((SBC, C), jnp.int32),
        pltpu.VMEM((SBC, C), jnp.int32),
        pltpu.VMEM((SBC, C), jnp.int32),
        pltpu.VMEM((C, D), jnp.float32),
        pltpu.VMEM((C, D), jnp.float32),
        pltpu.VMEM_SHARED((NSH, D), jnp.float32),
        pltpu.SemaphoreType.DMA,
        pltpu.SemaphoreType.DMA,
        pltpu.SemaphoreType.DMA,
        pltpu.SemaphoreType.DMA,
        pltpu.SemaphoreType.DMA,
        pltpu.SemaphoreType.DMA,
        pltpu.SemaphoreType.DMA,
    ],
)(_sc_body)


def _tc_body(agg_ref, x_ref, eps_ref, w1_ref, b1_ref, w2_ref, b2_ref,
             gamma_ref, beta_ref, out_ref, h_sc, s_sc):
    p = pl.program_id(0)
    j = pl.program_id(1)

    @pl.when(p == 0)
    def _pass0():
        @pl.when(j == 0)
        def _init():
            s_sc[...] = jnp.zeros_like(s_sc)

        a = agg_ref[0] + agg_ref[1] + eps_ref[0, 0] * x_ref[...]
        h1 = jnp.maximum(
            jnp.dot(a, w1_ref[...], preferred_element_type=jnp.float32)
            + b1_ref[...], 0.0)
        h2 = jnp.maximum(
            jnp.dot(h1, w2_ref[...], preferred_element_type=jnp.float32)
            + b2_ref[...], 0.0)
        h_sc[pl.ds(j * BR, BR), :] = h2
        s_sc[0:1, :] += jnp.sum(h2, axis=0, keepdims=True)
        s_sc[1:2, :] += jnp.sum(h2 * h2, axis=0, keepdims=True)

    @pl.when(p == 1)
    def _pass1():
        mean = s_sc[0:1, :] * (1.0 / N)
        var = s_sc[1:2, :] * (1.0 / N) - mean * mean
        inv = lax.rsqrt(var + 1e-5)
        hb = h_sc[pl.ds(j * BR, BR), :]
        out_ref[...] = gamma_ref[...] * ((hb - mean) * inv) + beta_ref[...]


def kernel(x, edge_index, eps, W1, b1, W2, b2, gamma, beta):
    src = edge_index[0].reshape(NW, E // NW)
    dst = edge_index[1].reshape(NW, E // NW)
    ppw = EPT - E // NW
    pad_ids = jnp.arange(NW * ppw, dtype=jnp.int32).reshape(NW, ppw)
    srcp = jnp.concatenate(
        [src, pad_ids % N], axis=1).reshape(NW, NSB, SBC, C)
    dstp = jnp.concatenate(
        [dst, N + pad_ids % NPAD_ROWS], axis=1).reshape(NW, NSB, SBC, C)
    zeros = jnp.zeros((NS, C, D), jnp.float32)

    agg2 = _sc_aggregate(x, srcp, dstp, zeros)

    def bsel(p, j):
        return jnp.where(p == 0, j, 0)

    out = pl.pallas_call(
        _tc_body,
        grid=(2, NBLK),
        in_specs=[
            pl.BlockSpec((NC, BR, D), lambda p, j: (0, bsel(p, j), 0)),
            pl.BlockSpec((BR, D), lambda p, j: (bsel(p, j), 0)),
            pl.BlockSpec(memory_space=pltpu.SMEM),
            pl.BlockSpec((D, H), lambda p, j: (0, 0)),
            pl.BlockSpec((1, H), lambda p, j: (0, 0)),
            pl.BlockSpec((H, D), lambda p, j: (0, 0)),
            pl.BlockSpec((1, D), lambda p, j: (0, 0)),
            pl.BlockSpec((1, D), lambda p, j: (0, 0)),
            pl.BlockSpec((1, D), lambda p, j: (0, 0)),
        ],
        out_specs=pl.BlockSpec((BR, D), lambda p, j: (jnp.where(p == 0, 0, j), 0)),
        out_shape=jax.ShapeDtypeStruct((N, D), jnp.float32),
        scratch_shapes=[
            pltpu.VMEM((N, D), jnp.float32),
            pltpu.VMEM((8, D), jnp.float32),
        ],
    )(agg2, x, eps.reshape(1, 1), W1, b1.reshape(1, H), W2,
      b2.reshape(1, D), gamma.reshape(1, D), beta.reshape(1, D))
    return out


# speedup vs baseline: 1.2095x; 1.2095x over previous
import functools

import jax
import jax.numpy as jnp
from jax import lax
from jax.experimental import pallas as pl
from jax.experimental.pallas import tpu as pltpu
from jax.experimental.pallas import tpu_sc as plsc

N = 10000
E = 320000
D = 128
H = 256

NC = 2
NS = 16
NW = NC * NS
C = 128
SBC = 20
EPT = -(-(E // NW) // (C * SBC)) * (C * SBC)
NCHUNK = EPT // C
NSB = NCHUNK // SBC
EPAD = EPT * NW
ZROWS = 632
NSH = ZROWS * NS
NPAD_ROWS = NSH - N
ORECS = 624
OREM = N - ORECS * NS
BR = 1000
NBLK = N // BR


def _sc_body(x_hbm, srcp_hbm, dstp_hbm, zeros_hbm, out_hbm,
             sidx_a, didx_a, sidx_b, didx_b, rows_v, rows1_v, agg_sh,
             gsem, gsem1, isem_a, isem_b, zsem):
    cid = lax.axis_index("c")
    sid = lax.axis_index("s")
    wid = cid * NS + sid

    pltpu.async_copy(srcp_hbm.at[wid, 0], sidx_a, isem_a)
    pltpu.async_copy(dstp_hbm.at[wid, 0], didx_a, isem_a)
    pltpu.async_copy(srcp_hbm.at[wid, 1], sidx_b, isem_b)
    pltpu.async_copy(dstp_hbm.at[wid, 1], didx_b, isem_b)

    base = sid * ZROWS
    zcopies = []
    pltpu.sync_copy(zeros_hbm.at[sid], rows1_v)
    for k in range(ZROWS // C):
        zcopies.append((rows1_v, agg_sh.at[pl.ds(base + k * C, C)]))
    rem = ZROWS % C
    if rem:
        zcopies.append((rows1_v.at[pl.ds(0, rem)],
                        agg_sh.at[pl.ds(base + (ZROWS // C) * C, rem)]))
    for s, d in zcopies:
        pltpu.async_copy(s, d, zsem)
    for s, d in zcopies:
        pltpu.make_async_copy(s, d, zsem).wait()
    plsc.subcore_barrier()

    def process(sidx_v, didx_v, isem, sb, restage, nsb):
        pltpu.make_async_copy(srcp_hbm.at[wid, sb], sidx_v, isem).wait()
        pltpu.make_async_copy(dstp_hbm.at[wid, sb], didx_v, isem).wait()
        pltpu.async_copy(x_hbm.at[sidx_v.at[0]], rows_v, gsem)
        if restage is not None:
            nsidx, ndidx, nisem = restage
            pltpu.async_copy(srcp_hbm.at[wid, nsb], nsidx, nisem)
            pltpu.async_copy(dstp_hbm.at[wid, nsb], ndidx, nisem)

        def chunk_pair(i, _):
            j = 2 * i
            pltpu.async_copy(x_hbm.at[sidx_v.at[j + 1]], rows1_v, gsem1)
            pltpu.make_async_copy(x_hbm.at[sidx_v.at[j]], rows_v, gsem).wait()
            pltpu.sync_copy(rows_v, agg_sh.at[didx_v.at[j]], add=True)

            @pl.when(j + 2 < SBC)
            def _next():
                pltpu.async_copy(x_hbm.at[sidx_v.at[j + 2]], rows_v, gsem)

            pltpu.make_async_copy(
                x_hbm.at[sidx_v.at[j + 1]], rows1_v, gsem1).wait()
            pltpu.sync_copy(rows1_v, agg_sh.at[didx_v.at[j + 1]], add=True)
            return _

        lax.fori_loop(0, SBC // 2, chunk_pair, None)

    banks = [(sidx_a, didx_a, isem_a), (sidx_b, didx_b, isem_b)]
    for sb in range(NSB):
        sidx_v, didx_v, isem = banks[sb % 2]
        if 1 <= sb <= NSB - 2:
            restage, nsb = banks[(sb + 1) % 2], sb + 1
        else:
            restage, nsb = None, None
        process(sidx_v, didx_v, isem, sb, restage, nsb)
    plsc.subcore_barrier()

    pltpu.sync_copy(agg_sh.at[pl.ds(sid * ORECS, ORECS)],
                    out_hbm.at[cid, pl.ds(sid * ORECS, ORECS)])

    @pl.when(sid == NS - 1)
    def _tail():
        pltpu.sync_copy(agg_sh.at[pl.ds(ORECS * NS, OREM)],
                        out_hbm.at[cid, pl.ds(ORECS * NS, OREM)])


_sc_aggregate = functools.partial(
    pl.kernel,
    out_type=jax.ShapeDtypeStruct((NC, N, D), jnp.float32),
    mesh=plsc.VectorSubcoreMesh(core_axis_name="c", subcore_axis_name="s"),
    scratch_types=[
        pltpu.VMEM((SBC, C), jnp.int32),
        pltpu.VMEM((SBC, C), jnp.int32),
        pltpu.VMEM((SBC, C), jnp.int32),
        pltpu.VMEM((SBC, C), jnp.int32),
        pltpu.VMEM((C, D), jnp.float32),
        pltpu.VMEM((C, D), jnp.float32),
        pltpu.VMEM_SHARED((NSH, D), jnp.float32),
        pltpu.SemaphoreType.DMA,
        pltpu.SemaphoreType.DMA,
        pltpu.SemaphoreType.DMA,
        pltpu.SemaphoreType.DMA,
        pltpu.SemaphoreType.DMA,
    ],
)(_sc_body)


def _tc_body(agg_ref, x_ref, eps_ref, w1_ref, b1_ref, w2_ref, b2_ref,
             gamma_ref, beta_ref, out_ref, h_sc, s_sc):
    p = pl.program_id(0)
    j = pl.program_id(1)

    @pl.when(p == 0)
    def _pass0():
        @pl.when(j == 0)
        def _init():
            s_sc[...] = jnp.zeros_like(s_sc)

        a = agg_ref[0] + agg_ref[1] + eps_ref[0, 0] * x_ref[...]
        h1 = jnp.maximum(
            jnp.dot(a, w1_ref[...], preferred_element_type=jnp.float32)
            + b1_ref[...], 0.0)
        h2 = jnp.maximum(
            jnp.dot(h1, w2_ref[...], preferred_element_type=jnp.float32)
            + b2_ref[...], 0.0)
        h_sc[pl.ds(j * BR, BR), :] = h2
        s_sc[0:1, :] += jnp.sum(h2, axis=0, keepdims=True)
        s_sc[1:2, :] += jnp.sum(h2 * h2, axis=0, keepdims=True)

    @pl.when(p == 1)
    def _pass1():
        mean = s_sc[0:1, :] * (1.0 / N)
        var = s_sc[1:2, :] * (1.0 / N) - mean * mean
        inv = lax.rsqrt(var + 1e-5)
        hb = h_sc[pl.ds(j * BR, BR), :]
        out_ref[...] = gamma_ref[...] * ((hb - mean) * inv) + beta_ref[...]


def kernel(x, edge_index, eps, W1, b1, W2, b2, gamma, beta):
    src = edge_index[0].reshape(NW, E // NW)
    dst = edge_index[1].reshape(NW, E // NW)
    ppw = EPT - E // NW
    pad_ids = jnp.arange(NW * ppw, dtype=jnp.int32).reshape(NW, ppw)
    srcp = jnp.concatenate(
        [src, pad_ids % N], axis=1).reshape(NW, NSB, SBC, C)
    dstp = jnp.concatenate(
        [dst, N + pad_ids % NPAD_ROWS], axis=1).reshape(NW, NSB, SBC, C)
    zeros = jnp.zeros((NS, C, D), jnp.float32)

    agg2 = _sc_aggregate(x, srcp, dstp, zeros)

    def bsel(p, j):
        return jnp.where(p == 0, j, 0)

    out = pl.pallas_call(
        _tc_body,
        grid=(2, NBLK),
        in_specs=[
            pl.BlockSpec((NC, BR, D), lambda p, j: (0, bsel(p, j), 0)),
            pl.BlockSpec((BR, D), lambda p, j: (bsel(p, j), 0)),
            pl.BlockSpec(memory_space=pltpu.SMEM),
            pl.BlockSpec((D, H), lambda p, j: (0, 0)),
            pl.BlockSpec((1, H), lambda p, j: (0, 0)),
            pl.BlockSpec((H, D), lambda p, j: (0, 0)),
            pl.BlockSpec((1, D), lambda p, j: (0, 0)),
            pl.BlockSpec((1, D), lambda p, j: (0, 0)),
            pl.BlockSpec((1, D), lambda p, j: (0, 0)),
        ],
        out_specs=pl.BlockSpec((BR, D), lambda p, j: (jnp.where(p == 0, 0, j), 0)),
        out_shape=jax.ShapeDtypeStruct((N, D), jnp.float32),
        scratch_shapes=[
            pltpu.VMEM((N, D), jnp.float32),
            pltpu.VMEM((8, D), jnp.float32),
        ],
    )(agg2, x, eps.reshape(1, 1), W1, b1.reshape(1, H), W2,
      b2.reshape(1, D), gamma.reshape(1, D), beta.reshape(1, D))
    return out
